# baseline (device time: 39166 ns/iter reference)
import jax
import jax.numpy as jnp
import numpy as np
from jax import lax
from jax.experimental import pallas as pl
from jax.experimental.pallas import tpu as pltpu

N_DEV = 8
B, SQ, D, DH, HL = 2, 128, 512, 64, 4
HD = HL * DH
R = B * SQ


def _consts():
    inv = 1.0 / (10000.0 ** (np.arange(0, DH, 2) / DH))
    pos = np.arange(SQ)[:, None] * inv[None, :]
    cos = np.repeat(np.cos(pos), 2, axis=-1)
    sin = np.repeat(np.sin(pos), 2, axis=-1)
    cos_t = np.tile(cos, (B, HL)).astype(np.float32)
    sin_t = np.tile(sin, (B, HL))
    even = (np.arange(HD) % 2 == 0)[None, :]
    sin_e = np.where(even, -sin_t, 0.0).astype(np.float32)
    sin_o = np.where(~even, sin_t, 0.0).astype(np.float32)
    blk = np.arange(R) // SQ
    mask = np.where(blk[:, None] == blk[None, :], 0.0, -1e9).astype(np.float32)
    return cos_t, sin_e, sin_o, mask


_CYCLE = [0, 1, 2, 3, 7, 6, 5, 4]
_NEXT = np.zeros(N_DEV, np.int32)
_PREV = np.zeros(N_DEV, np.int32)
for _i, _p in enumerate(_CYCLE):
    _NEXT[_p] = _CYCLE[(_i + 1) % N_DEV]
    _PREV[_p] = _CYCLE[(_i - 1) % N_DEV]


def kernel(x, Wq, Wk, Wv, Wo):
    cos_t, sin_e, sin_o, mask = _consts()
    bf = jnp.bfloat16

    def body(x_ref, wq_ref, wk_ref, wv_ref, wo_ref, cos_ref, sine_ref, sino_ref,
             mask_ref, out_ref,
             xb, ab_cw, ab_ccw,
             xs_cw_s, xs_cw_r, xs_ccw_s, xs_ccw_r,
             ac_cw_s, ac_cw_r, ac_ccw_s, ac_ccw_r):
        my = lax.axis_index("i")

        def lookup(table):
            r = jnp.int32(table[0])
            for i in range(1, N_DEV):
                r = jnp.where(my == i, jnp.int32(table[i]), r)
            return r

        right = lookup(_NEXT)
        left = lookup(_PREV)

        barrier = pltpu.get_barrier_semaphore()
        for nbr in (left, right):
            pl.semaphore_signal(barrier, inc=1, device_id=(nbr,),
                                device_id_type=pl.DeviceIdType.MESH)
        pl.semaphore_wait(barrier, 2)

        def x_cw(h, src=None):
            return pltpu.make_async_remote_copy(
                src_ref=xb.at[h - 1, 0] if src is None else src,
                dst_ref=xb.at[h, 0],
                send_sem=xs_cw_s.at[h - 1], recv_sem=xs_cw_r.at[h - 1],
                device_id=(right,), device_id_type=pl.DeviceIdType.MESH)

        def x_ccw(h, src=None):
            return pltpu.make_async_remote_copy(
                src_ref=xb.at[h - 1, 1] if src is None else src,
                dst_ref=xb.at[h, 1],
                send_sem=xs_ccw_s.at[h - 1], recv_sem=xs_ccw_r.at[h - 1],
                device_id=(left,), device_id_type=pl.DeviceIdType.MESH)

        def a_cw(h):
            return pltpu.make_async_remote_copy(
                src_ref=ab_cw.at[h - 1], dst_ref=ab_cw.at[h],
                send_sem=ac_cw_s.at[h - 1], recv_sem=ac_cw_r.at[h - 1],
                device_id=(right,), device_id_type=pl.DeviceIdType.MESH)

        def a_ccw(h):
            return pltpu.make_async_remote_copy(
                src_ref=ab_ccw.at[h - 1], dst_ref=ab_ccw.at[h],
                send_sem=ac_ccw_s.at[h - 1], recv_sem=ac_ccw_r.at[h - 1],
                device_id=(left,), device_id_type=pl.DeviceIdType.MESH)

        def rope(t):
            tm = jnp.concatenate([t[:, 1:], t[:, :1]], axis=1)
            tp = jnp.concatenate([t[:, -1:], t[:, :-1]], axis=1)
            return t * cos_ref[:, :] + tm * sine_ref[:, :] + tp * sino_ref[:, :]

        def contribution(xf2):
            q = rope(jnp.dot(xf2, wq_ref[:, :],
                             preferred_element_type=jnp.float32).astype(bf))
            k = rope(jnp.dot(xf2, wk_ref[:, :],
                             preferred_element_type=jnp.float32).astype(bf))
            v = jnp.dot(xf2, wv_ref[:, :],
                        preferred_element_type=jnp.float32).astype(bf)
            ctxs = []
            for hh in range(HL):
                sl = slice(hh * DH, (hh + 1) * DH)
                s = lax.dot_general(
                    q[:, sl], k[:, sl], (((1,), (1,)), ((), ())),
                    preferred_element_type=jnp.float32) * 0.125 + mask_ref[:, :]
                e = jnp.exp(s)
                r = jnp.sum(e, axis=1, keepdims=True)
                ctx = jnp.dot(e.astype(bf), v[:, sl],
                              preferred_element_type=jnp.float32) / r
                ctxs.append(ctx.astype(bf))
            return jnp.concatenate(ctxs, axis=1)

        def fold(h, ctx2, first_cw):

            def fold_cw():
                y = jnp.dot(ctx2[:SQ], wo_ref[:, :],
                            preferred_element_type=jnp.float32)
                a_cw(h).wait_recv()
                ab_cw[h] = (ab_cw[h].astype(jnp.float32) + y).astype(bf)
                a_cw(h + 1).start()

            def fold_ccw():
                y = jnp.dot(ctx2[SQ:], wo_ref[:, :],
                            preferred_element_type=jnp.float32)
                a_ccw(h).wait_recv()
                ab_ccw[h] = (ab_ccw[h].astype(jnp.float32) + y).astype(bf)
                a_ccw(h + 1).start()

            @pl.when(first_cw)
            def _():
                fold_cw()
                fold_ccw()

            @pl.when(jnp.logical_not(first_cw))
            def _():
                fold_ccw()
                fold_cw()

        x_cw(1, src=x_ref.at[0]).start()
        x_ccw(1, src=x_ref.at[1]).start()
        ctx2 = contribution(x_ref[:, :, :].reshape(R, D))
        ab_cw[0] = jnp.dot(ctx2[:SQ], wo_ref[:, :],
                           preferred_element_type=jnp.float32).astype(bf)
        a_cw(1).start()
        ab_ccw[0] = jnp.dot(ctx2[SQ:], wo_ref[:, :],
                            preferred_element_type=jnp.float32).astype(bf)
        a_ccw(1).start()
        x_cw(1).wait_recv()
        x_ccw(1).wait_recv()
        x_cw(2).start()
        x_ccw(2).start()
        ctx2 = contribution(xb[1].reshape(R, D))

        def hop(h, ctx2):
            fold(h, ctx2, h % 2 == 1)
            x_cw(h + 1).wait_recv()
            x_ccw(h + 1).wait_recv()

            @pl.when(h + 2 < N_DEV)
            def _():
                x_cw(h + 2).start()
                x_ccw(h + 2).start()

            ctx2 = contribution(xb[h + 1].reshape(R, D))

            x_cw(h).wait_send()
            x_ccw(h).wait_send()
            a_cw(h).wait_send()
            a_ccw(h).wait_send()
            return ctx2

        ctx2 = lax.fori_loop(1, N_DEV - 1, hop, ctx2)

        fold(N_DEV - 1, ctx2, jnp.bool_((N_DEV - 1) % 2 == 1))
        x_cw(N_DEV - 1).wait_send()
        x_ccw(N_DEV - 1).wait_send()
        a_cw(N_DEV - 1).wait_send()
        a_ccw(N_DEV - 1).wait_send()

        a_cw(N_DEV).wait_recv()
        out_ref[0] = ab_cw[N_DEV].astype(jnp.float32)
        a_ccw(N_DEV).wait_recv()
        out_ref[1] = ab_ccw[N_DEV].astype(jnp.float32)
        a_cw(N_DEV).wait_send()
        a_ccw(N_DEV).wait_send()

        def exit_barrier(sem):
            for nbr in (left, right):
                pl.semaphore_signal(sem, inc=1, device_id=(nbr,),
                                    device_id_type=pl.DeviceIdType.MESH)
            pl.semaphore_wait(sem, 2)

        pl.run_scoped(exit_barrier, pltpu.SemaphoreType.REGULAR)

    vmem = pl.BlockSpec(memory_space=pltpu.VMEM)
    return pl.pallas_call(
        body,
        out_shape=jax.ShapeDtypeStruct((B, SQ, D), jnp.float32),
        in_specs=[vmem] * 9,
        out_specs=vmem,
        scratch_shapes=[
            pltpu.VMEM((N_DEV, B, SQ, D), bf),
            pltpu.VMEM((N_DEV + 1, SQ, D), bf),
            pltpu.VMEM((N_DEV + 1, SQ, D), bf),
            pltpu.SemaphoreType.DMA((N_DEV,)),
            pltpu.SemaphoreType.DMA((N_DEV,)),
            pltpu.SemaphoreType.DMA((N_DEV,)),
            pltpu.SemaphoreType.DMA((N_DEV,)),
            pltpu.SemaphoreType.DMA((N_DEV,)),
            pltpu.SemaphoreType.DMA((N_DEV,)),
            pltpu.SemaphoreType.DMA((N_DEV,)),
            pltpu.SemaphoreType.DMA((N_DEV,)),
        ],
        compiler_params=pltpu.CompilerParams(collective_id=0),
    )(x.astype(bf), Wq.astype(bf), Wk.astype(bf), Wv.astype(bf), Wo.astype(bf),
      cos_t.astype(bf), sin_e.astype(bf), sin_o.astype(bf), mask)
